# baseline (speedup 1.0000x reference)
import functools

import jax
import jax.numpy as jnp
from jax import lax
from jax.experimental import pallas as pl
from jax.experimental.pallas import tpu as pltpu
from jax.experimental.pallas import tpu_sc as plsc

BATCH = 4
SEQ = 8192
DIM = 1024
SC_BATCH = 2
TC_BATCH = BATCH - SC_BATCH
CHUNK = 64
TC_BLOCK = 512


def _sc_body(table_hbm, out_hbm, buf, sem):
    info = plsc.get_sparse_core_info()
    nc, ns = info.num_cores, info.num_subcores
    nw = nc * ns
    rows_per_w = SEQ // nw
    wid = lax.axis_index("s") * nc + lax.axis_index("c")
    base = wid * rows_per_w

    for i in range(rows_per_w // CHUNK):
        row = base + i * CHUNK
        pltpu.sync_copy(table_hbm.at[pl.ds(row, CHUNK)], buf)
        for b in range(SC_BATCH):
            pltpu.sync_copy(buf, out_hbm.at[b, pl.ds(row, CHUNK)])


def _tc_body(table_ref, out_ref):
    rows = table_ref[...]
    for b in range(TC_BATCH):
        out_ref[b] = rows


@jax.jit
def _pos_embed(table):
    mesh = plsc.VectorSubcoreMesh(core_axis_name="c", subcore_axis_name="s")
    sc_fn = functools.partial(
        pl.kernel,
        mesh=mesh,
        out_type=jax.ShapeDtypeStruct((SC_BATCH, SEQ, DIM), jnp.float32),
        scratch_types=[
            pltpu.VMEM((CHUNK, DIM), jnp.float32),
            pltpu.SemaphoreType.DMA,
        ],
    )(_sc_body)
    sc_out = sc_fn(table)

    tc_out = pl.pallas_call(
        _tc_body,
        grid=(SEQ // TC_BLOCK,),
        in_specs=[pl.BlockSpec((TC_BLOCK, DIM), lambda i: (i, 0))],
        out_specs=pl.BlockSpec((TC_BATCH, TC_BLOCK, DIM), lambda i: (0, i, 0)),
        out_shape=jax.ShapeDtypeStruct((TC_BATCH, SEQ, DIM), jnp.float32),
    )(table)

    return jnp.concatenate([tc_out, sc_out], axis=0)


def kernel(inputs, table):
    del inputs
    return _pos_embed(table)


# speedup vs baseline: 2.3086x; 2.3086x over previous
import functools

import jax
import jax.numpy as jnp
from jax import lax
from jax.experimental import pallas as pl
from jax.experimental.pallas import tpu as pltpu
from jax.experimental.pallas import tpu_sc as plsc

BATCH = 4
SEQ = 8192
DIM = 1024
CHUNK = 64


def _pos_embed_kernel(table_hbm, out_hbm, buf, wsem):
    info = plsc.get_sparse_core_info()
    nc, ns = info.num_cores, info.num_subcores
    nw = nc * ns
    rows_per_w = SEQ // nw
    wid = lax.axis_index("s") * nc + lax.axis_index("c")
    base = wid * rows_per_w

    for i in range(rows_per_w // CHUNK):
        row = base + i * CHUNK
        pltpu.sync_copy(table_hbm.at[pl.ds(row, CHUNK)], buf)
        handles = [
            pltpu.async_copy(buf, out_hbm.at[b, pl.ds(row, CHUNK)], wsem)
            for b in range(BATCH)
        ]
        for h in handles:
            h.wait()


@jax.jit
def _pos_embed(table):
    mesh = plsc.VectorSubcoreMesh(core_axis_name="c", subcore_axis_name="s")
    fn = functools.partial(
        pl.kernel,
        mesh=mesh,
        out_type=jax.ShapeDtypeStruct((BATCH, SEQ, DIM), jnp.float32),
        scratch_types=[
            pltpu.VMEM((CHUNK, DIM), jnp.float32),
            pltpu.SemaphoreType.DMA,
        ],
    )(_pos_embed_kernel)
    return fn(table)


def kernel(inputs, table):
    del inputs
    return _pos_embed(table)
